# baseline (speedup 1.0000x reference)
import functools

import jax
import jax.numpy as jnp
from jax import lax
from jax.experimental import pallas as pl
from jax.experimental.pallas import tpu as pltpu
from jax.experimental.pallas import tpu_sc as plsc

NC = 2
NS = 16
NW = NC * NS
L = 16

D = 64
N_NEG = 20
C = 32
NROWS = C * N_NEG
NIDX_R = NROWS // 128
IDX_ROWS = 1 + NIDX_R


def _sc_dots(B):
  BPW = B // NW
  G = BPW // C

  mesh = plsc.VectorSubcoreMesh(
      core_axis_name="c", subcore_axis_name="s",
      num_cores=NC, num_subcores=NS)

  buf_types = [
      pltpu.VMEM((IDX_ROWS, 128), jnp.int32),
      pltpu.VMEM((C, D), jnp.float32),
      pltpu.VMEM((C, D), jnp.float32),
      pltpu.VMEM((NROWS, D), jnp.float32),
      pltpu.SemaphoreType.DMA,
      pltpu.SemaphoreType.DMA,
  ]

  @functools.partial(
      pl.kernel,
      out_type=[
          jax.ShapeDtypeStruct((B,), jnp.float32),
          jax.ShapeDtypeStruct((N_NEG * B,), jnp.float32),
      ],
      mesh=mesh,
      compiler_params=pltpu.CompilerParams(needs_layout_passes=False,
                                           use_tc_tiling_on_sc=False),
      scratch_types=buf_types + buf_types + [
          pltpu.VMEM((BPW,), jnp.float32),
          pltpu.VMEM((N_NEG * BPW,), jnp.float32),
      ],
  )
  def sc_kernel(idxpack_hbm, ctab_hbm, xtab_hbm, out1_hbm, out2_hbm,
                idx0, crow0, xrow0, nrow0, sem0, isem0,
                idx1, crow1, xrow1, nrow1, sem1, isem1,
                o1_v, o2_v):
    bufs = ((idx0, crow0, xrow0, nrow0, sem0, isem0),
            (idx1, crow1, xrow1, nrow1, sem1, isem1))
    wid = lax.axis_index("s") * NC + lax.axis_index("c")
    lanes = lax.iota(jnp.int32, L)

    def issue_idx(g, buf):
      idx, crow, xrow, nrow, sem, isem = bufs[buf]
      pltpu.async_copy(idxpack_hbm.at[wid * G + g], idx, isem)

    def wait_idx(buf):
      idx, crow, xrow, nrow, sem, isem = bufs[buf]
      pltpu.make_async_copy(idxpack_hbm.at[0], idx, isem).wait()

    def issue_rows(buf):
      idx, crow, xrow, nrow, sem, isem = bufs[buf]
      pltpu.async_copy(ctab_hbm.at[idx.at[0, pl.ds(0, C)]], crow, sem)
      pltpu.async_copy(xtab_hbm.at[idx.at[0, pl.ds(C, C)]], xrow, sem)
      for i in range(NIDX_R):
        pltpu.async_copy(xtab_hbm.at[idx.at[1 + i]],
                         nrow.at[pl.ds(i * 128, 128)], sem)

    def wait_rows(buf):
      idx, crow, xrow, nrow, sem, isem = bufs[buf]
      pltpu.make_async_copy(ctab_hbm.at[pl.ds(0, C)], crow, sem).wait()
      pltpu.make_async_copy(ctab_hbm.at[pl.ds(0, C)], xrow, sem).wait()
      pltpu.make_async_copy(ctab_hbm.at[pl.ds(0, NROWS)], nrow, sem).wait()

    def compute(g, buf):
      idx, crow, xrow, nrow, sem, isem = bufs[buf]
      for t in range(C // L):
        rows = lanes + t * L
        nr0 = rows * N_NEG

        def dbody(d, accs):
          dvec = jnp.zeros((L,), jnp.int32) + d
          cen = plsc.load_gather(crow, [rows, dvec])
          ctx = plsc.load_gather(xrow, [rows, dvec])
          new = [accs[0] + cen * ctx]
          for j in range(N_NEG):
            neg = plsc.load_gather(nrow, [nr0 + j, dvec])
            new.append(accs[j + 1] + cen * neg)
          return tuple(new)

        accs = lax.fori_loop(
            0, D, dbody,
            tuple(jnp.zeros((L,), jnp.float32) for _ in range(N_NEG + 1)))
        o1_v[pl.ds(g * C + t * L, L)] = accs[0]
        for j in range(N_NEG):
          o2_v[pl.ds(j * BPW + g * C + t * L, L)] = accs[j + 1]

    issue_idx(0, 0)
    wait_idx(0)
    issue_rows(0)
    issue_idx(1, 1)

    def pair(k, _):
      g0 = k * 2
      wait_rows(0)
      wait_idx(1)
      issue_rows(1)

      @pl.when(g0 + 2 < G)
      def _():
        issue_idx(g0 + 2, 0)

      compute(g0, 0)
      wait_rows(1)

      @pl.when(g0 + 2 < G)
      def _():
        wait_idx(0)
        issue_rows(0)

      @pl.when(g0 + 3 < G)
      def _():
        issue_idx(g0 + 3, 1)

      compute(g0 + 1, 1)
      return 0

    lax.fori_loop(0, G // 2, pair, 0)

    wbase = wid * BPW
    pltpu.sync_copy(o1_v, out1_hbm.at[pl.ds(wbase, BPW)])
    for j in range(N_NEG):
      pltpu.sync_copy(o2_v.at[pl.ds(j * BPW, BPW)],
                      out2_hbm.at[pl.ds(j * B + wbase, BPW)])

  return sc_kernel


def _tc_loss(d1, d2, total):
  def body(d1_ref, d2_ref, out_ref):
    x1 = d1_ref[...]
    x2 = -d2_ref[...]
    ls1 = jnp.minimum(x1, 0.0) - jnp.log(1.0 + jnp.exp(-jnp.abs(x1)))
    ls2 = jnp.minimum(x2, 0.0) - jnp.log(1.0 + jnp.exp(-jnp.abs(x2)))
    out_ref[0, 0] = -(jnp.sum(ls1) + jnp.sum(ls2)) / total

  out = pl.pallas_call(
      body,
      out_shape=jax.ShapeDtypeStruct((1, 1), jnp.float32),
      out_specs=pl.BlockSpec(memory_space=pltpu.SMEM),
  )(d1, d2)
  return out[0, 0]


def kernel(center, context, negative, center_table, context_table):
  B = center.shape[0]
  nchunks = B // C
  cen = center.reshape(nchunks, 1, C)
  ctx = context.reshape(nchunks, 1, C)
  pad = jnp.zeros((nchunks, 1, 128 - 2 * C), jnp.int32)
  row0 = jnp.concatenate([cen, ctx, pad], axis=2)
  negr = negative.reshape(nchunks, NIDX_R, 128)
  idxpack = jnp.concatenate([row0, negr], axis=1)
  dots1, dots2 = _sc_dots(B)(idxpack, center_table, context_table)
  return _tc_loss(dots1.reshape(B // 128, 128),
                  dots2.reshape(N_NEG * B // 128, 128),
                  float(B * (N_NEG + 1)))
